# baseline (speedup 1.0000x reference)
import functools

import jax
import jax.numpy as jnp
from jax import lax
from jax.experimental import pallas as pl
from jax.experimental.pallas import tpu as pltpu
from jax.experimental.pallas import tpu_sc as plsc

_NC = 2
_NS = 16
_NW = _NC * _NS
_CHUNK = 128
_TM = 1024


@functools.lru_cache(maxsize=None)
def _make_gather(bb: int, ss: int, d: int):
    ntok = bb * ss
    b_per_w = ntok // _NW
    nchunk = b_per_w // _CHUNK
    w_per_row = ss // b_per_w
    mesh = plsc.VectorSubcoreMesh(core_axis_name="c", subcore_axis_name="s")

    @functools.partial(
        pl.kernel,
        out_type=jax.ShapeDtypeStruct((ntok, d), jnp.float32),
        mesh=mesh,
        scratch_types=[
            pltpu.VMEM((b_per_w,), jnp.int32),
            pltpu.VMEM((b_per_w, d), jnp.float32),
            [pltpu.SemaphoreType.DMA for _ in range(nchunk)],
            pltpu.SemaphoreType.DMA,
        ],
    )
    def gather_kernel(idx_hbm, table_hbm, out_hbm, idx_v, rows_v, gsems, wsem):
        wid = lax.axis_index("s") * _NC + lax.axis_index("c")
        base = wid * b_per_w
        pltpu.sync_copy(
            idx_hbm.at[wid // w_per_row, pl.ds((wid % w_per_row) * b_per_w, b_per_w)],
            idx_v,
        )
        gathers = []
        for j in range(nchunk):
            gathers.append(
                pltpu.async_copy(
                    table_hbm.at[idx_v.at[pl.ds(j * _CHUNK, _CHUNK)]],
                    rows_v.at[pl.ds(j * _CHUNK, _CHUNK)],
                    gsems[j],
                )
            )
        writes = []
        for j in range(nchunk):
            gathers[j].wait()
            writes.append(
                pltpu.async_copy(
                    rows_v.at[pl.ds(j * _CHUNK, _CHUNK)],
                    out_hbm.at[pl.ds(base + j * _CHUNK, _CHUNK)],
                    wsem,
                )
            )
        for w in writes:
            w.wait()

    return gather_kernel


def _proj_body(x_ref, w_ref, s_ref, o_ref):
    o_ref[...] = (
        lax.dot_general(
            x_ref[...],
            w_ref[...],
            (((1,), (1,)), ((), ())),
            preferred_element_type=jnp.float32,
        )
        * s_ref[0]
    )


@functools.lru_cache(maxsize=None)
def _make_proj(ntok: int, d: int, m: int):
    grid = (ntok // _TM,)
    return pl.pallas_call(
        _proj_body,
        grid=grid,
        in_specs=[
            pl.BlockSpec((_TM, d), lambda i: (i, 0)),
            pl.BlockSpec((m, d), lambda i: (0, 0)),
            pl.BlockSpec(memory_space=pltpu.SMEM),
        ],
        out_specs=pl.BlockSpec((_TM, m), lambda i: (i, 0)),
        out_shape=jax.ShapeDtypeStruct((ntok, m), jnp.float32),
    )


def kernel(token_ids, embed_table, proj_weight, scale):
    b, s = token_ids.shape
    ntok = b * s
    d = embed_table.shape[1]
    m = proj_weight.shape[0]
    ids = token_ids.astype(jnp.int32)
    gathered = _make_gather(b, s, d)(ids, embed_table)
    out = _make_proj(ntok, d, m)(
        gathered, proj_weight, scale.astype(jnp.float32).reshape(1)
    )
    return out.reshape(b, s, m)


# speedup vs baseline: 1.0039x; 1.0039x over previous
import functools

import jax
import jax.numpy as jnp
from jax import lax
from jax.experimental import pallas as pl
from jax.experimental.pallas import tpu as pltpu
from jax.experimental.pallas import tpu_sc as plsc

_NC = 2
_NS = 16
_NW = _NC * _NS
_CHUNK = 128
_TM = 1024


@functools.lru_cache(maxsize=None)
def _make_gather(bb: int, ss: int, d: int):
    ntok = bb * ss
    b_per_w = ntok // _NW
    nchunk = b_per_w // _CHUNK
    w_per_row = ss // b_per_w
    mesh = plsc.VectorSubcoreMesh(core_axis_name="c", subcore_axis_name="s")

    @functools.partial(
        pl.kernel,
        out_type=jax.ShapeDtypeStruct((ntok, d), jnp.float32),
        mesh=mesh,
        scratch_types=[
            pltpu.VMEM((b_per_w,), jnp.int32),
            pltpu.VMEM((b_per_w, d), jnp.float32),
            [pltpu.SemaphoreType.DMA for _ in range(nchunk)],
            pltpu.SemaphoreType.DMA,
        ],
    )
    def gather_kernel(idx_hbm, table_hbm, out_hbm, idx_v, rows_v, gsems, wsem):
        wid = lax.axis_index("s") * _NC + lax.axis_index("c")
        base = wid * b_per_w
        pltpu.sync_copy(
            idx_hbm.at[wid // w_per_row, pl.ds((wid % w_per_row) * b_per_w, b_per_w)],
            idx_v,
        )
        gathers = []
        for j in range(nchunk):
            gathers.append(
                pltpu.async_copy(
                    table_hbm.at[idx_v.at[pl.ds(j * _CHUNK, _CHUNK)]],
                    rows_v.at[pl.ds(j * _CHUNK, _CHUNK)],
                    gsems[j],
                )
            )
        writes = []
        for j in range(nchunk):
            gathers[j].wait()
            writes.append(
                pltpu.async_copy(
                    rows_v.at[pl.ds(j * _CHUNK, _CHUNK)],
                    out_hbm.at[pl.ds(base + j * _CHUNK, _CHUNK)],
                    wsem,
                )
            )
        for w in writes:
            w.wait()

    return gather_kernel


def _proj_body(x_ref, w_ref, s_ref, o_ref):
    o_ref[...] = (
        lax.dot_general(
            x_ref[...].astype(jnp.bfloat16),
            w_ref[...].astype(jnp.bfloat16),
            (((1,), (1,)), ((), ())),
            preferred_element_type=jnp.float32,
        )
        * s_ref[0]
    )


@functools.lru_cache(maxsize=None)
def _make_proj(ntok: int, d: int, m: int):
    grid = (ntok // _TM,)
    return pl.pallas_call(
        _proj_body,
        grid=grid,
        in_specs=[
            pl.BlockSpec((_TM, d), lambda i: (i, 0)),
            pl.BlockSpec((m, d), lambda i: (0, 0)),
            pl.BlockSpec(memory_space=pltpu.SMEM),
        ],
        out_specs=pl.BlockSpec((_TM, m), lambda i: (i, 0)),
        out_shape=jax.ShapeDtypeStruct((ntok, m), jnp.float32),
    )


def kernel(token_ids, embed_table, proj_weight, scale):
    b, s = token_ids.shape
    ntok = b * s
    d = embed_table.shape[1]
    m = proj_weight.shape[0]
    ids = token_ids.astype(jnp.int32)
    gathered = _make_gather(b, s, d)(ids, embed_table)
    out = _make_proj(ntok, d, m)(
        gathered, proj_weight, scale.astype(jnp.float32).reshape(1)
    )
    return out.reshape(b, s, m)


# speedup vs baseline: 1.0044x; 1.0005x over previous
import functools

import jax
import jax.numpy as jnp
from jax import lax
from jax.experimental import pallas as pl
from jax.experimental.pallas import tpu as pltpu
from jax.experimental.pallas import tpu_sc as plsc

_NC = 2
_NS = 16
_NW = _NC * _NS
_CHUNK = 128
_TM = 1024


@functools.lru_cache(maxsize=None)
def _make_gather(bb: int, ss: int, d: int):
    ntok = bb * ss
    b_per_w = ntok // _NW
    nchunk = b_per_w // _CHUNK
    w_per_row = ss // b_per_w
    mesh = plsc.VectorSubcoreMesh(core_axis_name="c", subcore_axis_name="s")

    @functools.partial(
        pl.kernel,
        out_type=jax.ShapeDtypeStruct((ntok, d), jnp.float32),
        mesh=mesh,
        scratch_types=[
            pltpu.VMEM((b_per_w,), jnp.int32),
            pltpu.VMEM((b_per_w, d), jnp.float32),
            [pltpu.SemaphoreType.DMA for _ in range(nchunk)],
            pltpu.SemaphoreType.DMA,
        ],
    )
    def gather_kernel(idx_hbm, table_hbm, out_hbm, idx_v, rows_v, gsems, wsem):
        wid = lax.axis_index("s") * _NC + lax.axis_index("c")
        base = wid * b_per_w
        pltpu.sync_copy(
            idx_hbm.at[wid // w_per_row, pl.ds((wid % w_per_row) * b_per_w, b_per_w)],
            idx_v,
        )
        gathers = []
        for j in range(nchunk):
            gathers.append(
                pltpu.async_copy(
                    table_hbm.at[idx_v.at[pl.ds(j * _CHUNK, _CHUNK)]],
                    rows_v.at[pl.ds(j * _CHUNK, _CHUNK)],
                    gsems[j],
                )
            )
        writes = []
        for j in range(nchunk):
            gathers[j].wait()
            writes.append(
                pltpu.async_copy(
                    rows_v.at[pl.ds(j * _CHUNK, _CHUNK)],
                    out_hbm.at[pl.ds(base + j * _CHUNK, _CHUNK)],
                    wsem,
                )
            )
        for w in writes:
            w.wait()

    return gather_kernel


def _proj_body(x_ref, w_ref, s_ref, o_ref):
    o_ref[...] = (
        lax.dot_general(
            x_ref[...].astype(jnp.bfloat16),
            w_ref[...].astype(jnp.bfloat16),
            (((1,), (1,)), ((), ())),
            preferred_element_type=jnp.float32,
        )
        * s_ref[0]
    )


@functools.lru_cache(maxsize=None)
def _make_proj(ntok: int, d: int, m: int):
    grid = (ntok // _TM,)
    return pl.pallas_call(
        _proj_body,
        grid=grid,
        in_specs=[
            pl.BlockSpec((_TM, d), lambda i: (i, 0)),
            pl.BlockSpec((m, d), lambda i: (0, 0)),
            pl.BlockSpec(memory_space=pltpu.SMEM),
        ],
        out_specs=pl.BlockSpec((_TM, m), lambda i: (i, 0)),
        out_shape=jax.ShapeDtypeStruct((ntok, m), jnp.float32),
        compiler_params=pltpu.CompilerParams(
            dimension_semantics=("parallel",)
        ),
    )


def kernel(token_ids, embed_table, proj_weight, scale):
    b, s = token_ids.shape
    ntok = b * s
    d = embed_table.shape[1]
    m = proj_weight.shape[0]
    ids = token_ids.astype(jnp.int32)
    gathered = _make_gather(b, s, d)(ids, embed_table)
    out = _make_proj(ntok, d, m)(
        gathered, proj_weight, scale.astype(jnp.float32).reshape(1)
    )
    return out.reshape(b, s, m)
